# baseline (speedup 1.0000x reference)
import jax
import jax.numpy as jnp
from jax import lax
from jax.experimental import pallas as pl
from jax.experimental.pallas import tpu as pltpu
from jax.experimental.pallas import tpu_sc as plsc
from jax._src.pallas import mpmd


def kernel(input_ids, table):
    batch_size, seq_len = input_ids.shape
    max_len, d_model = table.shape

    info = plsc.get_sparse_core_info()
    nc, ns = info.num_cores, info.num_subcores
    nw = nc * ns
    scs_batches = 1
    tec_batches = batch_size - scs_batches

    rows_per_w = seq_len // nw
    chunk = 56
    nbuf = 2
    bounds = list(range(0, rows_per_w, chunk)) + [rows_per_w]
    sizes = [bounds[j + 1] - bounds[j] for j in range(len(bounds) - 1)]
    n_chunks = len(sizes)

    rows_per_c = seq_len // nc
    s_chunk = 128
    s_nbuf = 2
    sn_chunks = rows_per_c // s_chunk

    vec_mesh = plsc.VectorSubcoreMesh(core_axis_name="c", subcore_axis_name="s")
    scalar_mesh = plsc.ScalarSubcoreMesh(axis_name="c")

    def tec_fn(table_hbm, out_hbm, bufs, insem, outsem, sinsem, soutsem):
        del sinsem, soutsem
        wid = lax.axis_index("s") * nc + lax.axis_index("c")
        base = wid * rows_per_w

        def cp_in(i):
            start = base + bounds[i]
            return pltpu.async_copy(
                table_hbm.at[pl.ds(start, sizes[i])],
                bufs.at[i % nbuf, pl.ds(0, sizes[i])],
                insem,
            )

        def cp_out(i, b):
            start = base + bounds[i]
            return pltpu.async_copy(
                bufs.at[i % nbuf, pl.ds(0, sizes[i])],
                out_hbm.at[b, pl.ds(start, sizes[i])],
                outsem,
            )

        h_in = [None] * n_chunks
        h_out = [None] * n_chunks
        h_in[0] = cp_in(0)
        for i in range(n_chunks):
            if i + 1 < n_chunks:
                if i + 1 - nbuf >= 0:
                    for h in h_out[i + 1 - nbuf]:
                        h.wait()
                h_in[i + 1] = cp_in(i + 1)
            h_in[i].wait()
            h_out[i] = [cp_out(i, b) for b in range(tec_batches)]
        for i in range(max(0, n_chunks - nbuf), n_chunks):
            for h in h_out[i]:
                h.wait()

    def scs_fn(table_hbm, out_hbm, bufs, insem, outsem, sinsem, soutsem):
        del bufs, insem, outsem, sinsem
        base = lax.axis_index("c") * rows_per_c

        handles = []
        for b in range(scs_batches):
            handles.append(
                pltpu.async_copy(
                    table_hbm.at[pl.ds(base, rows_per_c)],
                    out_hbm.at[tec_batches + b, pl.ds(base, rows_per_c)],
                    soutsem,
                )
            )
        for h in handles:
            h.wait()

    call = mpmd.mpmd_map(
        [(scalar_mesh, scs_fn), (vec_mesh, tec_fn)],
        out_types=jax.ShapeDtypeStruct(
            (batch_size, seq_len, d_model), table.dtype
        ),
        scratch_types=(
            (pltpu.VMEM @ vec_mesh)((nbuf, chunk, d_model), jnp.float32),
            pltpu.SemaphoreType.DMA @ vec_mesh,
            pltpu.SemaphoreType.DMA @ vec_mesh,
            pltpu.SemaphoreType.DMA @ scalar_mesh,
            pltpu.SemaphoreType.DMA @ scalar_mesh,
        ),
    )
    return call(table)


# speedup vs baseline: 12.5890x; 12.5890x over previous
import jax
import jax.numpy as jnp
from jax import lax
from jax.experimental import pallas as pl
from jax.experimental.pallas import tpu as pltpu
from jax.experimental.pallas import tpu_sc as plsc
from jax._src.pallas import mpmd


def kernel(input_ids, table):
    batch_size, seq_len = input_ids.shape
    max_len, d_model = table.shape

    info = plsc.get_sparse_core_info()
    nc, ns = info.num_cores, info.num_subcores
    nw = nc * ns

    rows_per_w = seq_len // nw
    chunk = 56
    nbuf = 2
    bounds = list(range(0, rows_per_w, chunk)) + [rows_per_w]
    sizes = [bounds[j + 1] - bounds[j] for j in range(len(bounds) - 1)]
    n_chunks = len(sizes)

    m = 3
    s_nbuf = 4

    vec_mesh = plsc.VectorSubcoreMesh(core_axis_name="c", subcore_axis_name="s")
    scalar_mesh = plsc.ScalarSubcoreMesh(axis_name="c")

    def tec_fn(table_hbm, out_hbm, bufs, insem, outsem, sbufs, sinsem, soutsem):
        del sbufs, sinsem, soutsem
        wid = lax.axis_index("s") * nc + lax.axis_index("c")
        base = wid * rows_per_w

        def cp_in(i):
            start = base + bounds[i]
            return pltpu.async_copy(
                table_hbm.at[pl.ds(start, sizes[i])],
                bufs.at[i % nbuf, pl.ds(0, sizes[i])],
                insem,
            )

        def cp_out(i, b):
            start = base + bounds[i]
            return pltpu.async_copy(
                bufs.at[i % nbuf, pl.ds(0, sizes[i])],
                out_hbm.at[b, pl.ds(start, sizes[i])],
                outsem,
            )

        def outs(i):
            n_b = batch_size - 1 if i < m else batch_size
            return [cp_out(i, b) for b in range(n_b)]

        h_in = [None] * n_chunks
        h_out = [None] * n_chunks
        h_in[0] = cp_in(0)
        for i in range(n_chunks):
            if i + 1 < n_chunks:
                if i + 1 - nbuf >= 0:
                    for h in h_out[i + 1 - nbuf]:
                        h.wait()
                h_in[i + 1] = cp_in(i + 1)
            h_in[i].wait()
            h_out[i] = outs(i)
        for i in range(max(0, n_chunks - nbuf), n_chunks):
            for h in h_out[i]:
                h.wait()

    scs_rel_starts = [
        s * nc * rows_per_w + bounds[k] for s in range(ns) for k in range(m)
    ]
    sn_chunks = len(scs_rel_starts)

    def scs_fn(table_hbm, out_hbm, bufs, insem, outsem, sbufs, sinsem, soutsem):
        del bufs, insem, outsem
        core_off = lax.axis_index("c") * rows_per_w

        def cp_in(j):
            start = core_off + scs_rel_starts[j]
            return pltpu.async_copy(
                table_hbm.at[pl.ds(start, chunk)], sbufs.at[j % s_nbuf], sinsem
            )

        def cp_out(j):
            start = core_off + scs_rel_starts[j]
            return pltpu.async_copy(
                sbufs.at[j % s_nbuf],
                out_hbm.at[batch_size - 1, pl.ds(start, chunk)],
                soutsem,
            )

        h_in = [None] * sn_chunks
        h_out = [None] * sn_chunks
        h_in[0] = cp_in(0)
        for j in range(sn_chunks):
            if j + 1 < sn_chunks:
                if j + 1 - s_nbuf >= 0:
                    h_out[j + 1 - s_nbuf].wait()
                h_in[j + 1] = cp_in(j + 1)
            h_in[j].wait()
            h_out[j] = cp_out(j)
        for j in range(max(0, sn_chunks - s_nbuf), sn_chunks):
            h_out[j].wait()

    call = mpmd.mpmd_map(
        [(scalar_mesh, scs_fn), (vec_mesh, tec_fn)],
        out_types=jax.ShapeDtypeStruct(
            (batch_size, seq_len, d_model), table.dtype
        ),
        scratch_types=(
            (pltpu.VMEM @ vec_mesh)((nbuf, chunk, d_model), jnp.float32),
            pltpu.SemaphoreType.DMA @ vec_mesh,
            pltpu.SemaphoreType.DMA @ vec_mesh,
            pltpu.VMEM_SHARED((s_nbuf, chunk, d_model), jnp.float32),
            pltpu.SemaphoreType.DMA @ scalar_mesh,
            pltpu.SemaphoreType.DMA @ scalar_mesh,
        ),
    )
    return call(table)


# speedup vs baseline: 13.2501x; 1.0525x over previous
import jax
import jax.numpy as jnp
from jax import lax
from jax.experimental import pallas as pl
from jax.experimental.pallas import tpu as pltpu
from jax.experimental.pallas import tpu_sc as plsc
from jax._src.pallas import mpmd


def kernel(input_ids, table):
    batch_size, seq_len = input_ids.shape
    max_len, d_model = table.shape

    info = plsc.get_sparse_core_info()
    nc, ns = info.num_cores, info.num_subcores
    nw = nc * ns

    rows_per_w = seq_len // nw
    chunk = 56
    nbuf = 2
    bounds = list(range(0, rows_per_w, chunk)) + [rows_per_w]
    sizes = [bounds[j + 1] - bounds[j] for j in range(len(bounds) - 1)]
    n_chunks = len(sizes)

    m = 2
    s_nbuf = 2

    vec_mesh = plsc.VectorSubcoreMesh(core_axis_name="c", subcore_axis_name="s")
    scalar_mesh = plsc.ScalarSubcoreMesh(axis_name="c")

    def tec_fn(table_hbm, out_hbm, bufs, insem, outsem, sbufs, sinsem, soutsem):
        del sbufs, sinsem, soutsem
        wid = lax.axis_index("s") * nc + lax.axis_index("c")
        base = wid * rows_per_w

        def cp_in(i):
            start = base + bounds[i]
            return pltpu.async_copy(
                table_hbm.at[pl.ds(start, sizes[i])],
                bufs.at[i % nbuf, pl.ds(0, sizes[i])],
                insem,
            )

        def cp_out(i, b):
            start = base + bounds[i]
            return pltpu.async_copy(
                bufs.at[i % nbuf, pl.ds(0, sizes[i])],
                out_hbm.at[b, pl.ds(start, sizes[i])],
                outsem,
            )

        def outs(i):
            n_b = batch_size - 1 if i < m else batch_size
            return [cp_out(i, b) for b in range(n_b)]

        h_in = [None] * n_chunks
        h_out = [None] * n_chunks
        h_in[0] = cp_in(0)
        for i in range(n_chunks):
            if i + 1 < n_chunks:
                if i + 1 - nbuf >= 0:
                    for h in h_out[i + 1 - nbuf]:
                        h.wait()
                h_in[i + 1] = cp_in(i + 1)
            h_in[i].wait()
            h_out[i] = outs(i)
        for i in range(max(0, n_chunks - nbuf), n_chunks):
            for h in h_out[i]:
                h.wait()

    strip = bounds[m]
    scs_rel_starts = [s * nc * rows_per_w for s in range(ns)]
    sn_chunks = len(scs_rel_starts)

    def scs_fn(table_hbm, out_hbm, bufs, insem, outsem, sbufs, sinsem, soutsem):
        del bufs, insem, outsem
        core_off = lax.axis_index("c") * rows_per_w

        def cp_in(j):
            start = core_off + scs_rel_starts[j]
            return pltpu.async_copy(
                table_hbm.at[pl.ds(start, strip)], sbufs.at[j % s_nbuf], sinsem
            )

        def cp_out(j):
            start = core_off + scs_rel_starts[j]
            return pltpu.async_copy(
                sbufs.at[j % s_nbuf],
                out_hbm.at[batch_size - 1, pl.ds(start, strip)],
                soutsem,
            )

        h_in = [None] * sn_chunks
        h_out = [None] * sn_chunks
        h_in[0] = cp_in(0)
        for j in range(sn_chunks):
            if j + 1 < sn_chunks:
                if j + 1 - s_nbuf >= 0:
                    h_out[j + 1 - s_nbuf].wait()
                h_in[j + 1] = cp_in(j + 1)
            h_in[j].wait()
            h_out[j] = cp_out(j)
        for j in range(max(0, sn_chunks - s_nbuf), sn_chunks):
            h_out[j].wait()

    call = mpmd.mpmd_map(
        [(scalar_mesh, scs_fn), (vec_mesh, tec_fn)],
        out_types=jax.ShapeDtypeStruct(
            (batch_size, seq_len, d_model), table.dtype
        ),
        scratch_types=(
            (pltpu.VMEM @ vec_mesh)((nbuf, chunk, d_model), jnp.float32),
            pltpu.SemaphoreType.DMA @ vec_mesh,
            pltpu.SemaphoreType.DMA @ vec_mesh,
            pltpu.VMEM_SHARED((s_nbuf, strip, d_model), jnp.float32),
            pltpu.SemaphoreType.DMA @ scalar_mesh,
            pltpu.SemaphoreType.DMA @ scalar_mesh,
        ),
    )
    return call(table)


# speedup vs baseline: 14.2695x; 1.0769x over previous
import functools

import jax
import jax.numpy as jnp
from jax import lax
from jax.experimental import pallas as pl
from jax.experimental.pallas import tpu as pltpu
from jax.experimental.pallas import tpu_sc as plsc


def kernel(input_ids, table):
    batch_size, seq_len = input_ids.shape
    max_len, d_model = table.shape

    info = plsc.get_sparse_core_info()
    nc, ns = info.num_cores, info.num_subcores
    nw = nc * ns
    rows_per_w = seq_len // nw
    chunk = 56
    nbuf = 2
    bounds = list(range(0, rows_per_w, chunk)) + [rows_per_w]
    sizes = [bounds[j + 1] - bounds[j] for j in range(len(bounds) - 1)]
    n_chunks = len(sizes)

    mesh = plsc.VectorSubcoreMesh(core_axis_name="c", subcore_axis_name="s")

    @functools.partial(
        pl.kernel,
        mesh=mesh,
        out_type=jax.ShapeDtypeStruct((batch_size, seq_len, d_model), table.dtype),
        scratch_types=[
            pltpu.VMEM((nbuf, chunk, d_model), jnp.float32),
            pltpu.SemaphoreType.DMA,
            pltpu.SemaphoreType.DMA,
        ],
    )
    def sc_copy(table_hbm, out_hbm, bufs, insem, outsem):
        wid = lax.axis_index("s") * nc + lax.axis_index("c")
        base = wid * rows_per_w

        def cp_in(i):
            start = base + bounds[i]
            return pltpu.async_copy(
                table_hbm.at[pl.ds(start, sizes[i])],
                bufs.at[i % nbuf, pl.ds(0, sizes[i])],
                insem,
            )

        def cp_out(i, b):
            start = base + bounds[i]
            return pltpu.async_copy(
                bufs.at[i % nbuf, pl.ds(0, sizes[i])],
                out_hbm.at[b, pl.ds(start, sizes[i])],
                outsem,
            )

        h_in = [None] * n_chunks
        h_out = [None] * n_chunks
        h_in[0] = cp_in(0)
        for i in range(n_chunks):
            if i + 1 < n_chunks:
                if i + 1 - nbuf >= 0:
                    for h in h_out[i + 1 - nbuf]:
                        h.wait()
                h_in[i + 1] = cp_in(i + 1)
            h_in[i].wait()
            h_out[i] = [cp_out(i, b) for b in range(batch_size)]
        for i in range(max(0, n_chunks - nbuf), n_chunks):
            for h in h_out[i]:
                h.wait()

    return sc_copy(table)
